# Initial kernel scaffold; baseline (speedup 1.0000x reference)
import jax
import jax.numpy as jnp
from jax.experimental import pallas as pl


def kernel(boxes3D, boxes, classification, poses, confidence):
    raise NotImplementedError("write your pallas kernel here")


# speedup vs baseline: 143.9781x; 143.9781x over previous
import jax
import jax.numpy as jnp
from jax.experimental import pallas as pl

_NCLS = 8
_N = 1000
_NPAD = 1024
_G = _NCLS * _NPAD
_CHUNK = 128
_TOPK = 100
_NOUT = 128
_NEG = -1e30
_BIG = 99999.0
_KEEP = 11.0


def _fiota(shape, axis):
    return jax.lax.broadcasted_iota(jnp.int32, shape, axis).astype(jnp.float32)


def _class_body(acol_ref, arow_ref, p_ref, poses_ref, boxes_ref, scc_ref, scr_ref):
    A_c = acol_ref[0]
    A_r = arow_ref[0]
    P = p_ref[0]

    x1c, y1c, x2c, y2c = A_c[0:1, :], A_c[1:2, :], A_c[2:3, :], A_c[3:4, :]
    clsc, cfc = A_c[4:5, :], A_c[5:6, :]
    x1r, y1r, x2r, y2r = A_r[:, 0:1], A_r[:, 1:2], A_r[:, 2:3], A_r[:, 3:4]
    clsr, cfr, phr = A_r[:, 4:5], A_r[:, 5:6], A_r[:, 6:7]

    mx1 = jnp.maximum(x1r, x1c)
    my1 = jnp.maximum(y1r, y1c)
    mx2 = jnp.minimum(x2r, x2c)
    my2 = jnp.minimum(y2r, y2c)
    wid = mx2 - mx1 + 1.0
    hei = my2 - my1 + 1.0
    inter = wid * hei
    area_r = (x2r - x1r + 1.0) * (y2r - y1r + 1.0)
    area_c = (x2c - x1c + 1.0) * (y2c - y1c + 1.0)
    union = area_r + area_c - inter
    ov = jnp.where(union == 0.0, 0.0, inter / jnp.where(union == 0.0, 1.0, union))
    ov = jnp.where(wid <= 0.0, 0.0, ov)
    ov = jnp.where(hei <= 0.0, 0.0, ov)

    validr = clsr > 0.5
    validc = clsc > 0.5
    cond = (ov > 0.5) & validr & validc
    colio = _fiota( (_NPAD, _NPAD), 1)
    rowio = _fiota( (_NPAD, _NPAD), 0)

    jstar_r = jnp.min(jnp.where(cond, colio, _BIG), axis=1, keepdims=True)
    jstar_c = jnp.min(jnp.where(cond, rowio, _BIG), axis=0, keepdims=True)
    ovs_r = jnp.max(jnp.where(colio == jstar_r, ov, -1.0), axis=1, keepdims=True)
    ovs_c = jnp.max(jnp.where(rowio == jstar_c, ov, -1.0), axis=0, keepdims=True)
    key_r = (1.0 - ovs_r) * cfr
    key_c = (1.0 - ovs_c) * cfc
    mem_r = (jstar_r < _BIG) & (key_r != 0.0)
    mem_c = (jstar_c < _BIG) & (key_c != 0.0)

    samej = jstar_c == jstar_r
    less_cr = (key_c < key_r) | ((key_c == key_r) & (colio < rowio))
    rank_r = jnp.sum(jnp.where(samej & mem_c & less_cr, 1.0, 0.0), axis=1, keepdims=True)
    less_rc = (key_r < key_c) | ((key_r == key_c) & (rowio < colio))
    rank_c = jnp.sum(jnp.where(samej & mem_r & less_rc, 1.0, 0.0), axis=0, keepdims=True)
    sel_r = mem_r & (rank_r < _KEEP)
    sel_c = mem_c & (rank_c < _KEEP)

    W = jnp.where((rowio == jstar_c) & sel_c, 1.0, 0.0)
    Wt = jnp.where((colio == jstar_r) & sel_r, 1.0, 0.0)
    sums = jnp.dot(W, P, preferred_element_type=jnp.float32)
    d_r = jnp.sum(W, axis=1, keepdims=True)
    d_c = jnp.sum(Wt, axis=0, keepdims=True)
    den = jnp.where(d_r == 0.0, 1.0, d_r)
    zero = d_r == 0.0
    poses_ref[0] = jnp.where(zero, 0.0, sums[:, 0:12] / den)
    boxes_ref[0] = jnp.where(zero, 0.0, sums[:, 12:16] / den)

    anyv = jnp.max(jnp.where(validc, 1.0, 0.0))
    riota = _fiota( (_NPAD, 1), 0)
    ciota = _fiota( (1, _NPAD), 1)
    keep_r = (d_r > 0.0) & validr & (riota < float(_N))
    keep_c = (d_c > 0.0) & validc & (ciota < float(_N))
    sc_r = jnp.where(keep_r, clsr, _NEG)
    sc_c = jnp.where(keep_c, clsc, _NEG)
    ph_on = anyv == 0.0
    sc_r = jnp.where((riota == float(_N)) & ph_on, phr, sc_r)
    sc_c = jnp.where((ciota == float(_N)) & ph_on, A_c[6:7, :], sc_c)
    scr_ref[0] = sc_r
    scc_ref[0] = sc_c


def _rank_body(scol_ref, srow_ref, rank_ref):
    sc = scol_ref[0:1, :]
    sr = srow_ref[...]
    k = pl.program_id(0)
    rio = _fiota( (_CHUNK, 1), 0) + k.astype(jnp.float32) * float(_CHUNK)
    cio = _fiota( (_CHUNK, _G), 1)
    ahead = (sc > sr) | ((sc == sr) & (cio < rio))
    rank_ref[...] = jnp.sum(jnp.where(ahead, 1.0, 0.0), axis=1, keepdims=True)


def _select_body(rank_ref, srow_ref, poses_ref, boxes_ref,
                 ob_ref, os_ref, ol_ref, op_ref):
    rank = rank_ref[0:1, :]
    sio = _fiota( (_NOUT, _G), 0)
    R = jnp.where(rank == sio, 1.0, 0.0)

    g = _fiota( (_G, 1), 0)
    cid = jnp.floor(g / float(_NPAD))
    lane = g - cid * float(_NPAD)
    pad = lane >= float(_N)
    score = srow_ref[...]
    label = jnp.where(pad, -1.0, cid)
    bxs = jnp.where(pad, -1.0, boxes_ref[...])
    pss = jnp.where(pad, -1.0, poses_ref[...])
    payload = jnp.concatenate([bxs, pss, score, label], axis=1)
    out = jnp.dot(R, payload, preferred_element_type=jnp.float32)

    T = jnp.sum(jnp.where(score > -1e29, 1.0, 0.0))
    slot = _fiota( (_NOUT, 1), 0) < T
    ob_ref[...] = jnp.where(slot, out[:, 0:4], -1.0)
    op_ref[...] = jnp.where(slot, out[:, 4:16], -1.0)
    os_ref[...] = jnp.where(slot, out[:, 16:17], -1.0)
    ol_ref[...] = jnp.where(slot, out[:, 17:18], -1.0).astype(jnp.int32)


def kernel(boxes3D, boxes, classification, poses, confidence):
    del boxes3D
    f32 = jnp.float32
    bx = boxes.reshape(_N, _NCLS, 4).astype(f32)
    cls2 = classification.reshape(_N, _NCLS).astype(f32)
    cf2 = confidence.reshape(_N, _NCLS).astype(f32)
    ps2 = poses.reshape(_N, _NCLS, 12).astype(f32)
    ph_score = cls2[-1, -1]

    feat = jnp.stack(
        [bx[..., 0], bx[..., 1], bx[..., 2], bx[..., 3], cls2, cf2,
         jnp.broadcast_to(ph_score, (_N, _NCLS)), jnp.zeros((_N, _NCLS), f32)],
        axis=-1)
    feat = jnp.pad(feat, ((0, _NPAD - _N), (0, 0), (0, 0)))
    a_row = feat.transpose(1, 0, 2)
    a_col = feat.transpose(1, 2, 0)
    pmat = jnp.concatenate([ps2, bx], axis=-1)
    pmat = jnp.pad(pmat, ((0, _NPAD - _N), (0, 0), (0, 0))).transpose(1, 0, 2)

    poses_o, boxes_o, sc_c, sc_r = pl.pallas_call(
        _class_body,
        grid=(_NCLS,),
        in_specs=[
            pl.BlockSpec((1, 8, _NPAD), lambda c: (c, 0, 0)),
            pl.BlockSpec((1, _NPAD, 8), lambda c: (c, 0, 0)),
            pl.BlockSpec((1, _NPAD, 16), lambda c: (c, 0, 0)),
        ],
        out_specs=[
            pl.BlockSpec((1, _NPAD, 12), lambda c: (c, 0, 0)),
            pl.BlockSpec((1, _NPAD, 4), lambda c: (c, 0, 0)),
            pl.BlockSpec((1, 1, _NPAD), lambda c: (c, 0, 0)),
            pl.BlockSpec((1, _NPAD, 1), lambda c: (c, 0, 0)),
        ],
        out_shape=[
            jax.ShapeDtypeStruct((_NCLS, _NPAD, 12), f32),
            jax.ShapeDtypeStruct((_NCLS, _NPAD, 4), f32),
            jax.ShapeDtypeStruct((_NCLS, 1, _NPAD), f32),
            jax.ShapeDtypeStruct((_NCLS, _NPAD, 1), f32),
        ],
    )(a_col, a_row, pmat)

    s_col = sc_c.reshape(1, _G)
    s_row = sc_r.reshape(_G, 1)

    rank = pl.pallas_call(
        _rank_body,
        grid=(_G // _CHUNK,),
        in_specs=[
            pl.BlockSpec((1, _G), lambda k: (0, 0)),
            pl.BlockSpec((_CHUNK, 1), lambda k: (k, 0)),
        ],
        out_specs=pl.BlockSpec((_CHUNK, 1), lambda k: (k, 0)),
        out_shape=jax.ShapeDtypeStruct((_G, 1), f32),
    )(s_col, s_row)

    boxes_f, scores_f, labels_f, poses_f = pl.pallas_call(
        _select_body,
        in_specs=[
            pl.BlockSpec((1, _G), lambda: (0, 0)),
            pl.BlockSpec((_G, 1), lambda: (0, 0)),
            pl.BlockSpec((_G, 12), lambda: (0, 0)),
            pl.BlockSpec((_G, 4), lambda: (0, 0)),
        ],
        out_specs=[
            pl.BlockSpec((_NOUT, 4), lambda: (0, 0)),
            pl.BlockSpec((_NOUT, 1), lambda: (0, 0)),
            pl.BlockSpec((_NOUT, 1), lambda: (0, 0)),
            pl.BlockSpec((_NOUT, 12), lambda: (0, 0)),
        ],
        out_shape=[
            jax.ShapeDtypeStruct((_NOUT, 4), f32),
            jax.ShapeDtypeStruct((_NOUT, 1), f32),
            jax.ShapeDtypeStruct((_NOUT, 1), jnp.int32),
            jax.ShapeDtypeStruct((_NOUT, 12), f32),
        ],
    )(rank.reshape(1, _G), s_row, poses_o.reshape(_G, 12), boxes_o.reshape(_G, 4))

    return (boxes_f[:_TOPK], scores_f[:_TOPK, 0], labels_f[:_TOPK, 0],
            poses_f[:_TOPK])
